# baseline (speedup 1.0000x reference)
import functools

import jax
import jax.numpy as jnp
from jax import lax
from jax.experimental import pallas as pl
from jax.experimental.pallas import tpu as pltpu
from jax.experimental.pallas import tpu_sc as plsc

D = 32
NC = 2
NS = 16
NW = NC * NS
C = 8


def _gather3d(x, table):
    b, f = x.shape
    per_w = b // NW
    n_chunks = per_w // C
    mesh = plsc.VectorSubcoreMesh(core_axis_name="c", subcore_axis_name="s")

    @functools.partial(
        pl.kernel,
        mesh=mesh,
        out_type=jax.ShapeDtypeStruct((b, f, D), jnp.float32),
        scratch_types=[
            pltpu.VMEM((per_w, f), jnp.int32),
            pltpu.VMEM((2, C, f, D), jnp.float32),
            pltpu.SemaphoreType.DMA,
            pltpu.SemaphoreType.DMA,
        ],
        compiler_params=pltpu.CompilerParams(use_tc_tiling_on_sc=False),
    )
    def k(x_hbm, table_hbm, out_hbm, idx_v, rows_v, sem_g, sem_o):
        wid = lax.axis_index("s") * NC + lax.axis_index("c")
        base = wid * per_w

        pltpu.sync_copy(x_hbm.at[pl.ds(base, per_w)], idx_v)

        def fire_gather(c, bf):
            for r in range(C):
                pltpu.async_copy(
                    table_hbm.at[idx_v.at[c * C + r]],
                    rows_v.at[bf].at[r],
                    sem_g,
                )

        def drain_gather(bf):
            pltpu.make_async_copy(
                out_hbm.at[pl.ds(0, C)], rows_v.at[bf], sem_g
            ).wait()

        def fire_out(c, bf):
            pltpu.async_copy(
                rows_v.at[bf], out_hbm.at[pl.ds(base + c * C, C)], sem_o
            )

        def wait_out(bf):
            pltpu.make_async_copy(
                rows_v.at[bf], out_hbm.at[pl.ds(base, C)], sem_o
            ).wait()

        fire_gather(0, 0)
        drain_gather(0)
        fire_gather(1, 1)
        fire_out(0, 0)

        def body(t, carry):
            c = 2 * t + 1
            drain_gather(1)
            wait_out(0)
            fire_gather(c + 1, 0)
            fire_out(c, 1)
            drain_gather(0)
            wait_out(1)
            fire_gather(c + 2, 1)
            fire_out(c + 1, 0)
            return carry

        lax.fori_loop(0, (n_chunks - 2) // 2, body, 0)

        drain_gather(1)
        wait_out(0)
        fire_out(n_chunks - 1, 1)
        wait_out(1)

    return k(x, table)


def kernel(x, table):
    return _gather3d(x.astype(jnp.int32), table)


# speedup vs baseline: 1.0001x; 1.0001x over previous
import functools

import jax
import jax.numpy as jnp
from jax import lax
from jax.experimental import pallas as pl
from jax.experimental.pallas import tpu as pltpu
from jax.experimental.pallas import tpu_sc as plsc

D = 32
NC = 2
NS = 16
NW = NC * NS
C = 8


def _gather3d(x, table):
    b, f = x.shape
    per_w = b // NW
    n_chunks = per_w // C
    mesh = plsc.VectorSubcoreMesh(core_axis_name="c", subcore_axis_name="s")

    @functools.partial(
        pl.kernel,
        mesh=mesh,
        out_type=jax.ShapeDtypeStruct((b, f, D), jnp.float32),
        scratch_types=[
            pltpu.VMEM((per_w, f), jnp.int32),
            pltpu.VMEM((2, C, f, D), jnp.float32),
            pltpu.SemaphoreType.DMA,
            pltpu.SemaphoreType.DMA,
        ],
        compiler_params=pltpu.CompilerParams(use_tc_tiling_on_sc=False),
    )
    def k(x_hbm, table_hbm, out_hbm, idx_v, rows_v, sem_g, sem_o):
        wid = lax.axis_index("s") * NC + lax.axis_index("c")
        base = wid * per_w

        pltpu.sync_copy(x_hbm.at[pl.ds(base, per_w)], idx_v)

        def fire_gather(c, bf):
            for r in range(C):
                pltpu.async_copy(
                    table_hbm.at[idx_v.at[c * C + r]],
                    rows_v.at[bf].at[r],
                    sem_g,
                )

        def drain_gather(bf):
            pltpu.make_async_copy(
                out_hbm.at[pl.ds(0, C)], rows_v.at[bf], sem_g
            ).wait()

        def fire_out(c, bf):
            pltpu.async_copy(
                rows_v.at[bf], out_hbm.at[pl.ds(base + c * C, C)], sem_o
            )

        def wait_out(bf):
            pltpu.make_async_copy(
                rows_v.at[bf], out_hbm.at[pl.ds(base, C)], sem_o
            ).wait()

        fire_gather(0, 0)
        drain_gather(0)
        fire_gather(1, 1)
        fire_out(0, 0)

        def body(t, carry):
            c = 2 * t + 1
            drain_gather(1)
            wait_out(0)
            fire_gather(c + 1, 0)
            fire_out(c, 1)
            drain_gather(0)
            wait_out(1)
            fire_gather(c + 2, 1)
            fire_out(c + 1, 0)
            return carry

        lax.fori_loop(0, (n_chunks - 2) // 2, body, 0)

        drain_gather(1)
        wait_out(0)
        fire_out(n_chunks - 1, 1)
        wait_out(1)

    return k(x, table)


def kernel(x, table):
    out = _gather3d(x.astype(jnp.int32), table * jnp.float32(1.0))
    return out * jnp.float32(1.0)


# speedup vs baseline: 1.0362x; 1.0361x over previous
import functools

import jax
import jax.numpy as jnp
from jax import lax
from jax.experimental import pallas as pl
from jax.experimental.pallas import tpu as pltpu
from jax.experimental.pallas import tpu_sc as plsc

D = 32
NC = 2
NS = 16
NW = NC * NS
G = 128
K = 13
CHUNK = G * K


def _flat_gather(idx2d, table):
    n_rows, _ = idx2d.shape
    n = n_rows * G
    b_per_w = n // NW
    n_chunks = b_per_w // CHUNK
    mesh = plsc.VectorSubcoreMesh(core_axis_name="c", subcore_axis_name="s")

    @functools.partial(
        pl.kernel,
        mesh=mesh,
        out_type=jax.ShapeDtypeStruct((n, D), jnp.float32),
        scratch_types=[
            pltpu.VMEM((K, G), jnp.int32),
            pltpu.VMEM((CHUNK, D), jnp.float32),
            pltpu.SemaphoreType.DMA,
        ],
        compiler_params=pltpu.CompilerParams(use_tc_tiling_on_sc=False),
    )
    def k(idx_hbm, table_hbm, out_hbm, idx_v, rows_v, sem):
        wid = lax.axis_index("s") * NC + lax.axis_index("c")
        base = wid * b_per_w

        def body(i, carry):
            off = base + i * CHUNK
            pltpu.sync_copy(idx_hbm.at[pl.ds(off // G, K)], idx_v)
            copies = [
                pltpu.async_copy(
                    table_hbm.at[idx_v.at[j]],
                    rows_v.at[pl.ds(j * G, G)],
                    sem,
                )
                for j in range(K)
            ]
            for c in copies:
                c.wait()
            pltpu.sync_copy(rows_v, out_hbm.at[pl.ds(off, CHUNK)])
            return carry

        lax.fori_loop(0, n_chunks, body, 0)

    return k(idx2d, table)


def kernel(x, table):
    b, f = x.shape
    idx2d = x.reshape(b * f // G, G).astype(jnp.int32)
    out = _flat_gather(idx2d, table)
    return out.reshape(b, f, D)
